# baseline (speedup 1.0000x reference)
import functools

import jax
import jax.numpy as jnp
from jax import lax
from jax.experimental import pallas as pl
from jax.experimental.pallas import tpu as pltpu
from jax.experimental.pallas import tpu_sc as plsc


_CH = 32
_RING = 3


def _build_sc_kernel(B, S, D):
    info = plsc.get_sparse_core_info()
    NC, NS = info.num_cores, info.num_subcores
    NW = NC * NS
    SEQ_W = S // NW
    NCH = SEQ_W // _CH
    UNITS = NCH * B
    CW = _CH * D
    NV = CW // 16

    mesh = plsc.VectorSubcoreMesh(core_axis_name="c", subcore_axis_name="s")

    NCHT = B * S * D // CW

    @functools.partial(
        pl.kernel,
        mesh=mesh,
        out_type=jax.ShapeDtypeStruct((NCHT, CW), jnp.float32),
        scratch_types=(
            [pltpu.VMEM((1, CW), jnp.float32) for _ in range(_RING + 2)]
            + [
                pltpu.SemaphoreType.DMA((_RING,)),
                pltpu.SemaphoreType.DMA((2,)),
                pltpu.SemaphoreType.DMA((_RING,)),
            ]
        ),
    )
    def sc_kernel(x_hbm, pe_hbm, out_hbm, bx0, bx1, bx2, bp0, bp1,
                  x_sem, pe_sem, out_sem):
        bufx = [bx0, bx1, bx2]
        bufp = [bp0, bp1]
        wid = lax.axis_index("s") * NC + lax.axis_index("c")
        seq0 = wid * SEQ_W

        def chunk_row(u):
            c, b = u // B, u % B
            return (b * S + seq0) // _CH + c

        def x_copy(u):
            slot = u % _RING
            return pltpu.async_copy(
                x_hbm.at[pl.ds(chunk_row(u), 1), :], bufx[slot],
                x_sem.at[slot])

        def pe_copy(c):
            return pltpu.async_copy(
                pe_hbm.at[pl.ds(seq0 // _CH + c, 1), :], bufp[c % 2],
                pe_sem.at[c % 2])

        def out_copy(u):
            slot = u % _RING
            return pltpu.async_copy(
                bufx[slot], out_hbm.at[pl.ds(chunk_row(u), 1), :],
                out_sem.at[slot])

        def add_pe(u):
            slot, c2 = u % _RING, (u // B) % 2

            def body(k, _):
                sl = pl.ds(k * 16, 16)
                plsc.addupdate(bufx[slot].at[0, sl], bufp[c2][0, sl])
                return 0

            lax.fori_loop(0, NV, body, 0, unroll=8)

        pe_d, x_d, out_d = {}, {}, {}
        pe_d[0] = pe_copy(0)
        x_d[0] = x_copy(0)
        for u in range(UNITS):
            nxt = u + 1
            if nxt < UNITS:
                if nxt >= _RING:
                    out_d[nxt - _RING].wait()
                x_d[nxt] = x_copy(nxt)
                if nxt % B == 0 and nxt // B < NCH:
                    pe_d[nxt // B] = pe_copy(nxt // B)
            if u % B == 0:
                pe_d[u // B].wait()
            x_d[u].wait()
            add_pe(u)
            out_d[u] = out_copy(u)
        for u in range(max(0, UNITS - _RING), UNITS):
            out_d[u].wait()

    return sc_kernel


def kernel(x, pe_table):
    B, S, D = x.shape
    pe = pe_table[:S]
    cw = _CH * D
    xf = x.reshape(B * S * D // cw, cw)
    out = _build_sc_kernel(B, S, D)(xf, pe.reshape(S * D // cw, cw))
    return out.reshape(B, S, D)


# speedup vs baseline: 4.4773x; 4.4773x over previous
import jax
import jax.numpy as jnp
from jax.experimental import pallas as pl
from jax.experimental.pallas import tpu as pltpu


_R = 1024
_K = 6
_P = 4


def _make_body(C, NP, R, D):
    def body(x_ref, pe_ref, o_ref, xbuf, pecache, insem, pesem, outsem):
        def in_copy(t):
            return pltpu.make_async_copy(
                x_ref.at[pl.ds(t * R, R), :], xbuf.at[t % _K], insem.at[t % _K])

        def out_copy(t):
            return pltpu.make_async_copy(
                xbuf.at[t % _K], o_ref.at[pl.ds(t * R, R), :], outsem.at[t % _K])

        pe_copies = [
            pltpu.make_async_copy(
                pe_ref.at[pl.ds(p * R, R), :], pecache.at[p], pesem.at[p])
            for p in range(NP)
        ]
        for c in pe_copies:
            c.start()
        for j in range(min(_P, C)):
            in_copy(j).start()

        out_waited = [False] * C
        pe_waited = [False] * NP
        for t in range(C):
            slot = t % _K
            in_copy(t).wait()
            p = t % NP
            if not pe_waited[p]:
                pe_copies[p].wait()
                pe_waited[p] = True
            xbuf[slot] = xbuf[slot] + pecache[p]
            out_copy(t).start()
            j = t + _P
            if j < C:
                if j >= _K:
                    out_copy(j - _K).wait()
                    out_waited[j - _K] = True
                in_copy(j).start()
        for t in range(C):
            if not out_waited[t]:
                out_copy(t).wait()

    return body


def kernel(x, pe_table):
    B, S, D = x.shape
    pe = pe_table[:S]
    xf = x.reshape(B * S, D)
    R = _R if (B * S) % _R == 0 and S % _R == 0 else S
    C = (B * S) // R
    NP = S // R
    out = pl.pallas_call(
        _make_body(C, NP, R, D),
        in_specs=[
            pl.BlockSpec(memory_space=pl.ANY),
            pl.BlockSpec(memory_space=pl.ANY),
        ],
        out_specs=pl.BlockSpec(memory_space=pl.ANY),
        out_shape=jax.ShapeDtypeStruct((B * S, D), x.dtype),
        scratch_shapes=[
            pltpu.VMEM((_K, R, D), x.dtype),
            pltpu.VMEM((NP, R, D), x.dtype),
            pltpu.SemaphoreType.DMA((_K,)),
            pltpu.SemaphoreType.DMA((NP,)),
            pltpu.SemaphoreType.DMA((_K,)),
        ],
    )(xf, pe)
    return out.reshape(B, S, D)


# speedup vs baseline: 4.4945x; 1.0038x over previous
import jax
import jax.numpy as jnp
from jax.experimental import pallas as pl
from jax.experimental.pallas import tpu as pltpu


_R = 1024
_K = 6
_P = 4


def _make_body(C, NP, R, D):
    def body(x_ref, pe_ref, o_ref, xbuf, pecache, insem, pesem, outsem):
        def in_copy(t):
            return pltpu.make_async_copy(
                x_ref.at[pl.ds(t * R, R), :], xbuf.at[t % _K], insem.at[t % _K])

        def out_copy(t):
            return pltpu.make_async_copy(
                xbuf.at[t % _K], o_ref.at[pl.ds(t * R, R), :], outsem.at[t % _K])

        pe_copies = [
            pltpu.make_async_copy(
                pe_ref.at[pl.ds(p * R, R), :], pecache.at[p], pesem.at[p])
            for p in range(NP)
        ]
        in_copy(0).start()
        pe_copies[0].start()
        for j in range(1, min(_P, C)):
            in_copy(j).start()
        for c in pe_copies[1:]:
            c.start()

        out_waited = [False] * C
        pe_waited = [False] * NP
        for t in range(C):
            slot = t % _K
            in_copy(t).wait()
            p = t % NP
            if not pe_waited[p]:
                pe_copies[p].wait()
                pe_waited[p] = True
            xbuf[slot] = xbuf[slot] + pecache[p]
            out_copy(t).start()
            j = t + _P
            if j < C:
                if j >= _K:
                    out_copy(j - _K).wait()
                    out_waited[j - _K] = True
                in_copy(j).start()
        for t in range(C):
            if not out_waited[t]:
                out_copy(t).wait()

    return body


def kernel(x, pe_table):
    B, S, D = x.shape
    pe = pe_table[:S]
    xf = x.reshape(B * S, D)
    R = _R if (B * S) % _R == 0 and S % _R == 0 else S
    C = (B * S) // R
    NP = S // R
    out = pl.pallas_call(
        _make_body(C, NP, R, D),
        in_specs=[
            pl.BlockSpec(memory_space=pl.ANY),
            pl.BlockSpec(memory_space=pl.ANY),
        ],
        out_specs=pl.BlockSpec(memory_space=pl.ANY),
        out_shape=jax.ShapeDtypeStruct((B * S, D), x.dtype),
        scratch_shapes=[
            pltpu.VMEM((_K, R, D), x.dtype),
            pltpu.VMEM((NP, R, D), x.dtype),
            pltpu.SemaphoreType.DMA((_K,)),
            pltpu.SemaphoreType.DMA((NP,)),
            pltpu.SemaphoreType.DMA((_K,)),
        ],
    )(xf, pe)
    return out.reshape(B, S, D)
